# Initial kernel scaffold; baseline (speedup 1.0000x reference)
import functools

import jax
import jax.numpy as jnp
from jax import lax
from jax.experimental import pallas as pl
from jax.experimental.pallas import tpu as pltpu
from jax.experimental.pallas import tpu_sc as plsc

NC, NS, LANES = 2, 16, 16
NW = NC * NS
LP = 208
GCH = 104
NV = LP // LANES


def _make_sc_call(B, V):
    rows_per_w = B // NW
    groups = rows_per_w // LANES
    vpad = V + 8
    sent = V

    def body(ids_hbm, w_hbm, b_hbm, out_hbm,
             ids_v, vals_v, buf_v, flat_v, out_v, b_v, gsem):
        wid = lax.axis_index("s") * NC + lax.axis_index("c")
        base = wid * rows_per_w
        pltpu.sync_copy(b_hbm, b_v)
        pltpu.sync_copy(ids_hbm.at[pl.ds(base, rows_per_w)], ids_v)
        iota = lax.iota(jnp.int32, LANES)

        def do_row(i, _):
            cp0 = pltpu.async_copy(
                w_hbm.at[ids_v.at[i, pl.ds(0, GCH)]],
                vals_v.at[pl.ds(0, GCH)], gsem)
            cp1 = pltpu.async_copy(
                w_hbm.at[ids_v.at[i, pl.ds(GCH, GCH)]],
                vals_v.at[pl.ds(GCH, GCH)], gsem)
            for k in range(NV):
                idv = ids_v[i, pl.ds(k * LANES, LANES)]
                plsc.store_scatter(buf_v, [idv], iota + (k * LANES))
            cp0.wait()
            cp1.wait()
            acc = jnp.zeros((LANES,), jnp.float32)
            for k in range(NV):
                idv = ids_v[i, pl.ds(k * LANES, LANES)]
                back = plsc.load_gather(buf_v, [idv])
                keep = back == (iota + k * LANES)
                vals = vals_v[pl.ds(k * LANES, LANES)]
                acc = acc + jnp.where(keep, vals, 0.0)
            flat_v[pl.ds((i % LANES) * LANES, LANES)] = acc
            return 0

        for g in range(groups):
            lax.fori_loop(g * LANES, (g + 1) * LANES, do_row, 0)
            tot = jnp.zeros((LANES,), jnp.float32)
            for l in range(LANES):
                tot = tot + plsc.load_gather(flat_v, [iota * LANES + l])
            logits = tot + b_v[...]
            out_v[pl.ds(g * LANES, LANES)] = 1.0 / (1.0 + jnp.exp(-logits))
        pltpu.sync_copy(out_v, out_hbm.at[pl.ds(base, rows_per_w)])

    call = functools.partial(
        pl.kernel,
        out_type=jax.ShapeDtypeStruct((B,), jnp.float32),
        mesh=plsc.VectorSubcoreMesh(
            core_axis_name="c", subcore_axis_name="s",
            num_cores=NC, num_subcores=NS),
        scratch_types=[
            pltpu.VMEM((rows_per_w, LP), jnp.int32),
            pltpu.VMEM((LP,), jnp.float32),
            pltpu.VMEM((vpad,), jnp.int32),
            pltpu.VMEM((LANES * LANES,), jnp.float32),
            pltpu.VMEM((rows_per_w,), jnp.float32),
            pltpu.VMEM((LANES,), jnp.float32),
            pltpu.SemaphoreType.DMA,
        ],
    )(body)
    return call, sent, vpad


def kernel(token_ids, W, b):
    B, L = token_ids.shape
    V = W.shape[1]
    call, sent, vpad = _make_sc_call(B, V)
    ids = token_ids.astype(jnp.int32)
    ids_pad = jnp.pad(ids, ((0, 0), (0, LP - L)), constant_values=sent)
    w_pad = jnp.pad(W.reshape(-1).astype(jnp.float32), (0, vpad - V))
    b16 = jnp.broadcast_to(b.astype(jnp.float32).reshape(1), (LANES,))
    out = call(ids_pad, w_pad, b16)
    return out.reshape(B, 1)


# speedup vs baseline: 4.9699x; 4.9699x over previous
import functools

import jax
import jax.numpy as jnp
from jax import lax
from jax.experimental import pallas as pl
from jax.experimental.pallas import tpu as pltpu
from jax.experimental.pallas import tpu_sc as plsc

NC, NS, LANES = 2, 16, 16
NW = NC * NS
LP = 256
GCH = 128
NV = LP // LANES


def _make_sc_call(B, V):
    rows_per_w = B // NW
    groups = rows_per_w // LANES
    vpad = V + 8
    sent = V

    def body(ids_hbm, w_hbm, b_hbm, out_hbm,
             ids_v, vals_v, buf_v, flat_v, out_v, b_v, gsem):
        wid = lax.axis_index("s") * NC + lax.axis_index("c")
        base = wid * rows_per_w
        pltpu.sync_copy(b_hbm, b_v)
        pltpu.sync_copy(
            ids_hbm.at[pl.ds(pl.multiple_of(base * LP, LP), rows_per_w * LP)],
            ids_v)
        iota = lax.iota(jnp.int32, LANES)

        def do_row(i, _):
            roff = pl.multiple_of(i * LP, LP)
            cp0 = pltpu.async_copy(
                w_hbm.at[ids_v.at[pl.ds(roff, GCH)]],
                vals_v.at[pl.ds(0, GCH)], gsem)
            cp1 = pltpu.async_copy(
                w_hbm.at[ids_v.at[pl.ds(roff + GCH, GCH)]],
                vals_v.at[pl.ds(GCH, GCH)], gsem)
            for k in range(NV):
                idv = ids_v[pl.ds(roff + k * LANES, LANES)]
                plsc.store_scatter(buf_v, [idv], iota + (k * LANES))
            cp0.wait()
            cp1.wait()
            acc = jnp.zeros((LANES,), jnp.float32)
            for k in range(NV):
                idv = ids_v[pl.ds(roff + k * LANES, LANES)]
                back = plsc.load_gather(buf_v, [idv])
                keep = back == (iota + k * LANES)
                vals = vals_v[pl.ds(k * LANES, LANES)]
                acc = acc + jnp.where(keep, vals, 0.0)
            foff = pl.multiple_of(lax.rem(i, LANES) * LANES, LANES)
            flat_v[pl.ds(foff, LANES)] = acc
            return 0

        for g in range(groups):
            lax.fori_loop(g * LANES, (g + 1) * LANES, do_row, 0)
            tot = jnp.zeros((LANES,), jnp.float32)
            for l in range(LANES):
                tot = tot + plsc.load_gather(flat_v, [iota * LANES + l])
            logits = tot + b_v[...]
            out_v[pl.ds(g * LANES, LANES)] = 1.0 / (1.0 + jnp.exp(-logits))
        pltpu.sync_copy(
            out_v,
            out_hbm.at[pl.ds(pl.multiple_of(base, rows_per_w), rows_per_w)])

    call = functools.partial(
        pl.kernel,
        out_type=jax.ShapeDtypeStruct((B,), jnp.float32),
        mesh=plsc.VectorSubcoreMesh(
            core_axis_name="c", subcore_axis_name="s",
            num_cores=NC, num_subcores=NS),
        compiler_params=pltpu.CompilerParams(needs_layout_passes=False),
        scratch_types=[
            pltpu.VMEM((rows_per_w * LP,), jnp.int32),
            pltpu.VMEM((LP,), jnp.float32),
            pltpu.VMEM((vpad,), jnp.int32),
            pltpu.VMEM((LANES * LANES,), jnp.float32),
            pltpu.VMEM((rows_per_w,), jnp.float32),
            pltpu.VMEM((LANES,), jnp.float32),
            pltpu.SemaphoreType.DMA,
        ],
    )(body)
    return call, sent, vpad


def kernel(token_ids, W, b):
    B, L = token_ids.shape
    V = W.shape[1]
    call, sent, vpad = _make_sc_call(B, V)
    ids = token_ids.astype(jnp.int32)
    ids_pad = jnp.pad(ids, ((0, 0), (0, LP - L)), constant_values=sent)
    w_pad = jnp.pad(W.reshape(-1).astype(jnp.float32), (0, vpad - V))
    b16 = jnp.broadcast_to(b.astype(jnp.float32).reshape(1), (LANES,))
    out = call(ids_pad.reshape(-1), w_pad, b16)
    return out.reshape(B, 1)


# speedup vs baseline: 21.5922x; 4.3446x over previous
import functools

import jax
import jax.numpy as jnp
from jax import lax
from jax.experimental import pallas as pl
from jax.experimental.pallas import tpu as pltpu
from jax.experimental.pallas import tpu_sc as plsc

NC, NS, LANES = 2, 16, 16
NW = NC * NS
LP = 208
GCH = 104
NV = LP // LANES


def _make_sc_call(B, V):
    rows_per_w = B // NW
    groups = rows_per_w // LANES
    vpad = V + 8
    sent = V

    def body(ids_hbm, w_hbm, b_hbm, out_hbm,
             ids_v, vals_v, buf_v, flat_v, out_v, b_v, gsem):
        wid = lax.axis_index("s") * NC + lax.axis_index("c")
        base = wid * rows_per_w
        pltpu.sync_copy(b_hbm, b_v)
        pltpu.sync_copy(
            ids_hbm.at[pl.ds(pl.multiple_of(base * LP, LP), rows_per_w * LP)],
            ids_v)
        iota = lax.iota(jnp.int32, LANES)

        cps = []
        for i in range(rows_per_w):
            for c in range(LP // GCH):
                off = i * LP + c * GCH
                cps.append(pltpu.async_copy(
                    w_hbm.at[ids_v.at[pl.ds(off, GCH)]],
                    vals_v.at[pl.ds(off, GCH)], gsem))
        for cp in cps:
            cp.wait()

        def do_row(i, _):
            roff = pl.multiple_of(i * LP, LP)
            for k in range(NV):
                idv = ids_v[pl.ds(roff + k * LANES, LANES)]
                plsc.store_scatter(buf_v, [idv], iota + (k * LANES))
            acc = jnp.zeros((LANES,), jnp.float32)
            for k in range(NV):
                idv = ids_v[pl.ds(roff + k * LANES, LANES)]
                back = plsc.load_gather(buf_v, [idv])
                keep = back == (iota + k * LANES)
                vals = vals_v[pl.ds(roff + k * LANES, LANES)]
                acc = acc + jnp.where(keep, vals, 0.0)
            foff = pl.multiple_of(lax.rem(i, LANES) * LANES, LANES)
            flat_v[pl.ds(foff, LANES)] = acc
            return 0

        for g in range(groups):
            lax.fori_loop(g * LANES, (g + 1) * LANES, do_row, 0)
            tot = jnp.zeros((LANES,), jnp.float32)
            for l in range(LANES):
                tot = tot + plsc.load_gather(flat_v, [iota * LANES + l])
            logits = tot + b_v[...]
            out_v[pl.ds(g * LANES, LANES)] = 1.0 / (1.0 + jnp.exp(-logits))
        pltpu.sync_copy(
            out_v,
            out_hbm.at[pl.ds(pl.multiple_of(base, rows_per_w), rows_per_w)])

    call = functools.partial(
        pl.kernel,
        out_type=jax.ShapeDtypeStruct((B,), jnp.float32),
        mesh=plsc.VectorSubcoreMesh(
            core_axis_name="c", subcore_axis_name="s",
            num_cores=NC, num_subcores=NS),
        compiler_params=pltpu.CompilerParams(needs_layout_passes=False),
        scratch_types=[
            pltpu.VMEM((rows_per_w * LP,), jnp.int32),
            pltpu.VMEM((rows_per_w * LP,), jnp.float32),
            pltpu.VMEM((vpad,), jnp.int32),
            pltpu.VMEM((LANES * LANES,), jnp.float32),
            pltpu.VMEM((rows_per_w,), jnp.float32),
            pltpu.VMEM((LANES,), jnp.float32),
            pltpu.SemaphoreType.DMA,
        ],
    )(body)
    return call, sent, vpad


def kernel(token_ids, W, b):
    B, L = token_ids.shape
    V = W.shape[1]
    call, sent, vpad = _make_sc_call(B, V)
    ids = token_ids.astype(jnp.int32)
    ids_pad = jnp.pad(ids, ((0, 0), (0, LP - L)), constant_values=sent)
    w_pad = jnp.pad(W.reshape(-1).astype(jnp.float32), (0, vpad - V))
    b16 = jnp.broadcast_to(b.astype(jnp.float32).reshape(1), (LANES,))
    out = call(ids_pad.reshape(-1), w_pad, b16)
    return out.reshape(B, 1)


# speedup vs baseline: 21.6269x; 1.0016x over previous
import functools

import jax
import jax.numpy as jnp
from jax import lax
from jax.experimental import pallas as pl
from jax.experimental.pallas import tpu as pltpu
from jax.experimental.pallas import tpu_sc as plsc

NC, NS, LANES = 2, 16, 16
NW = NC * NS
LP = 208
GCH = 104
NV = LP // LANES


def _make_sc_call(B, V):
    rows_per_w = B // NW
    groups = rows_per_w // LANES
    vpad = V + 8
    sent = V

    def body(ids_hbm, w_hbm, b_hbm, out_hbm,
             ids_v, vals_v, buf_v, flat_v, out_v, b_v, gsem):
        wid = lax.axis_index("s") * NC + lax.axis_index("c")
        base = wid * rows_per_w
        pltpu.sync_copy(b_hbm, b_v)
        pltpu.sync_copy(
            ids_hbm.at[pl.ds(pl.multiple_of(base * LP, LP), rows_per_w * LP)],
            ids_v)
        iota = lax.iota(jnp.int32, LANES)

        with jax.named_scope("fire"):
            cps = []
            for i in range(rows_per_w):
                for c in range(LP // GCH):
                    off = i * LP + c * GCH
                    cps.append(pltpu.async_copy(
                        w_hbm.at[ids_v.at[pl.ds(off, GCH)]],
                        vals_v.at[pl.ds(off, GCH)], gsem))
        with jax.named_scope("drain"):
            for cp in cps:
                cp.wait()

        def do_row(i, _):
            roff = pl.multiple_of(i * LP, LP)
            for k in range(NV):
                idv = ids_v[pl.ds(roff + k * LANES, LANES)]
                plsc.store_scatter(buf_v, [idv], iota + (k * LANES))
            acc = jnp.zeros((LANES,), jnp.float32)
            for k in range(NV):
                idv = ids_v[pl.ds(roff + k * LANES, LANES)]
                back = plsc.load_gather(buf_v, [idv])
                keep = back == (iota + k * LANES)
                vals = vals_v[pl.ds(roff + k * LANES, LANES)]
                acc = acc + jnp.where(keep, vals, 0.0)
            foff = pl.multiple_of(lax.rem(i, LANES) * LANES, LANES)
            flat_v[pl.ds(foff, LANES)] = acc
            return 0

        with jax.named_scope("compute"):
            for g in range(groups):
                lax.fori_loop(g * LANES, (g + 1) * LANES, do_row, 0)
                tot = jnp.zeros((LANES,), jnp.float32)
                for l in range(LANES):
                    tot = tot + plsc.load_gather(flat_v, [iota * LANES + l])
                logits = tot + b_v[...]
                out_v[pl.ds(g * LANES, LANES)] = 1.0 / (1.0 + jnp.exp(-logits))
        pltpu.sync_copy(
            out_v,
            out_hbm.at[pl.ds(pl.multiple_of(base, rows_per_w), rows_per_w)])

    call = functools.partial(
        pl.kernel,
        out_type=jax.ShapeDtypeStruct((B,), jnp.float32),
        mesh=plsc.VectorSubcoreMesh(
            core_axis_name="c", subcore_axis_name="s",
            num_cores=NC, num_subcores=NS),
        compiler_params=pltpu.CompilerParams(needs_layout_passes=False),
        scratch_types=[
            pltpu.VMEM((rows_per_w * LP,), jnp.int32),
            pltpu.VMEM((rows_per_w * LP,), jnp.float32),
            pltpu.VMEM((vpad,), jnp.int32),
            pltpu.VMEM((LANES * LANES,), jnp.float32),
            pltpu.VMEM((rows_per_w,), jnp.float32),
            pltpu.VMEM((LANES,), jnp.float32),
            pltpu.SemaphoreType.DMA,
        ],
    )(body)
    return call, sent, vpad


def kernel(token_ids, W, b):
    B, L = token_ids.shape
    V = W.shape[1]
    call, sent, vpad = _make_sc_call(B, V)
    ids = token_ids.astype(jnp.int32)
    ids_pad = jnp.pad(ids, ((0, 0), (0, LP - L)), constant_values=sent)
    w_pad = jnp.pad(W.reshape(-1).astype(jnp.float32), (0, vpad - V))
    b16 = jnp.broadcast_to(b.astype(jnp.float32).reshape(1), (LANES,))
    out = call(ids_pad.reshape(-1), w_pad, b16)
    return out.reshape(B, 1)


# speedup vs baseline: 42.8403x; 1.9809x over previous
import functools

import jax
import jax.numpy as jnp
from jax import lax
from jax.experimental import pallas as pl
from jax.experimental.pallas import tpu as pltpu
from jax.experimental.pallas import tpu_sc as plsc

NC, NS, LANES = 2, 16, 16
NW = NC * NS
LP = 208
GCH = 128
NV = LP // LANES


def _make_sc_call(B, V):
    rows_per_w = B // NW
    groups = rows_per_w // LANES
    nwords = rows_per_w * LP
    nch = nwords // GCH
    vpad = V + 8
    sent = V

    def body(ids_hbm, w_hbm, b_hbm, out_hbm,
             ids_v, vals_v, buf_v, flat_v, out_v, b_v, w_sh, gsem):
        wid = lax.axis_index("s") * NC + lax.axis_index("c")
        base = wid * rows_per_w
        pltpu.sync_copy(b_hbm, b_v)
        pltpu.sync_copy(
            ids_hbm.at[pl.ds(pl.multiple_of(base * LP, LP), nwords)],
            ids_v)
        @pl.when(lax.axis_index("s") == 0)
        def _():
            pltpu.sync_copy(w_hbm, w_sh)
        plsc.subcore_barrier()
        iota = lax.iota(jnp.int32, LANES)

        with jax.named_scope("fire"):
            cps = []
            for c in range(nch):
                off = c * GCH
                cps.append(pltpu.async_copy(
                    w_sh.at[ids_v.at[pl.ds(off, GCH)]],
                    vals_v.at[pl.ds(off, GCH)], gsem))

        with jax.named_scope("compute"):
            drained = 0
            for i in range(rows_per_w):
                need = -(-((i + 1) * LP) // GCH)
                while drained < need:
                    cps[drained].wait()
                    drained += 1
                roff = i * LP
                for k in range(NV):
                    idv = ids_v[pl.ds(roff + k * LANES, LANES)]
                    plsc.store_scatter(buf_v, [idv], iota + (k * LANES))
                acc = jnp.zeros((LANES,), jnp.float32)
                for k in range(NV):
                    idv = ids_v[pl.ds(roff + k * LANES, LANES)]
                    back = plsc.load_gather(buf_v, [idv])
                    keep = back == (iota + k * LANES)
                    vals = vals_v[pl.ds(roff + k * LANES, LANES)]
                    acc = acc + jnp.where(keep, vals, 0.0)
                flat_v[pl.ds((i % LANES) * LANES, LANES)] = acc
                if i % LANES == LANES - 1:
                    tot = jnp.zeros((LANES,), jnp.float32)
                    for l in range(LANES):
                        tot = tot + plsc.load_gather(flat_v, [iota * LANES + l])
                    logits = tot + b_v[...]
                    g = i // LANES
                    out_v[pl.ds(g * LANES, LANES)] = (
                        1.0 / (1.0 + jnp.exp(-logits)))
        pltpu.sync_copy(
            out_v,
            out_hbm.at[pl.ds(pl.multiple_of(base, rows_per_w), rows_per_w)])

    call = functools.partial(
        pl.kernel,
        out_type=jax.ShapeDtypeStruct((B,), jnp.float32),
        mesh=plsc.VectorSubcoreMesh(
            core_axis_name="c", subcore_axis_name="s",
            num_cores=NC, num_subcores=NS),
        compiler_params=pltpu.CompilerParams(needs_layout_passes=False),
        scratch_types=[
            pltpu.VMEM((nwords,), jnp.int32),
            pltpu.VMEM((nwords,), jnp.float32),
            pltpu.VMEM((vpad,), jnp.int32),
            pltpu.VMEM((LANES * LANES,), jnp.float32),
            pltpu.VMEM((rows_per_w,), jnp.float32),
            pltpu.VMEM((LANES,), jnp.float32),
            pltpu.VMEM_SHARED((vpad,), jnp.float32),
            pltpu.SemaphoreType.DMA,
        ],
    )(body)
    return call, sent, vpad


def kernel(token_ids, W, b):
    B, L = token_ids.shape
    V = W.shape[1]
    call, sent, vpad = _make_sc_call(B, V)
    ids = token_ids.astype(jnp.int32)
    ids_pad = jnp.pad(ids, ((0, 0), (0, LP - L)), constant_values=sent)
    w_pad = jnp.pad(W.reshape(-1).astype(jnp.float32), (0, vpad - V))
    b16 = jnp.broadcast_to(b.astype(jnp.float32).reshape(1), (LANES,))
    out = call(ids_pad.reshape(-1), w_pad, b16)
    return out.reshape(B, 1)
